# Initial kernel scaffold; baseline (speedup 1.0000x reference)
import jax
import jax.numpy as jnp
from jax.experimental import pallas as pl


def kernel(y_pred):
    raise NotImplementedError("write your pallas kernel here")


# speedup vs baseline: 41.3355x; 41.3355x over previous
import functools

import jax
import jax.numpy as jnp
from jax import lax
from jax.experimental import pallas as pl
from jax.experimental.pallas import tpu as pltpu
from jax.experimental.pallas import tpu_sc as plsc

B = 32
N = 1048576
BINS = 256
NCOPY = 16
CH = 32768
NCHUNK = N // CH
VREGS = CH // 16


def _minmax_body(x_ref, mn_ref, scale_ref):
    x = x_ref[...]
    mn = jnp.min(x)
    mx = jnp.max(x)
    rng = mx - mn
    safe = jnp.where(rng > 0, rng, jnp.float32(1.0))
    scale = jnp.float32(256.0) / safe
    mn_ref[...] = jnp.full((1, 1, 128), mn, jnp.float32)
    scale_ref[...] = jnp.full((1, 1, 128), scale, jnp.float32)


def _minmax(y3):
    return pl.pallas_call(
        _minmax_body,
        grid=(B,),
        in_specs=[pl.BlockSpec((1, N // 128, 128), lambda i: (i, 0, 0))],
        out_specs=[
            pl.BlockSpec((1, 1, 128), lambda i: (i, 0, 0)),
            pl.BlockSpec((1, 1, 128), lambda i: (i, 0, 0)),
        ],
        out_shape=[
            jax.ShapeDtypeStruct((B, 1, 128), jnp.float32),
            jax.ShapeDtypeStruct((B, 1, 128), jnp.float32),
        ],
    )(y3)


_mesh = plsc.VectorSubcoreMesh(core_axis_name="c", subcore_axis_name="s")


@functools.partial(
    pl.kernel,
    mesh=_mesh,
    compiler_params=pltpu.CompilerParams(needs_layout_passes=False),
    out_type=jax.ShapeDtypeStruct((B, BINS), jnp.float32),
    scratch_types=[
        pltpu.VMEM((CH,), jnp.float32),
        pltpu.VMEM((CH,), jnp.float32),
        pltpu.VMEM((BINS * NCOPY,), jnp.float32),
        pltpu.VMEM((BINS,), jnp.float32),
        pltpu.VMEM((16,), jnp.float32),
        pltpu.VMEM((16,), jnp.float32),
        pltpu.SemaphoreType.DMA,
        pltpu.SemaphoreType.DMA,
    ],
)
def _hist_sc(y_hbm, mn_hbm, scale_hbm, out_hbm,
             buf0, buf1, hist, outv, mnv, scv, sem0, sem1):
    row = lax.axis_index("s") * 2 + lax.axis_index("c")

    pltpu.sync_copy(mn_hbm.at[row, pl.ds(0, 16)], mnv)
    pltpu.sync_copy(scale_hbm.at[row, pl.ds(0, 16)], scv)

    zeros16 = jnp.zeros((16,), jnp.float32)

    def zbody(i, carry):
        hist[pl.ds(i * 16, 16)] = zeros16
        return carry

    lax.fori_loop(0, (BINS * NCOPY) // 16, zbody, 0)

    mn_v = mnv[...]
    sc_v = scv[...]
    lane_base = lax.iota(jnp.int32, 16) * 256
    ones = jnp.ones((16,), jnp.float32)
    cap = jnp.full((16,), 255, jnp.int32)

    bufs = (buf0, buf1)
    sems = (sem0, sem1)

    def chunk_body(buf, _):
        def body(i, carry):
            x = buf[pl.ds(i * 16, 16)]
            t = (x - mn_v) * sc_v
            idx = jnp.minimum(t.astype(jnp.int32), cap)
            plsc.addupdate_scatter(hist, [idx + lane_base], ones)
            return carry

        lax.fori_loop(0, VREGS, body, 0)

    handles = [pltpu.async_copy(y_hbm.at[row, pl.ds(0, CH)], buf0, sem0)]
    for c in range(NCHUNK):
        handles[c].wait()
        if c + 1 < NCHUNK:
            handles.append(
                pltpu.async_copy(
                    y_hbm.at[row, pl.ds((c + 1) * CH, CH)],
                    bufs[(c + 1) % 2],
                    sems[(c + 1) % 2],
                )
            )
        chunk_body(bufs[c % 2], None)

    for g in range(BINS // 16):
        acc = hist[pl.ds(g * 16, 16)]
        for cpy in range(1, NCOPY):
            acc = acc + hist[pl.ds(cpy * BINS + g * 16, 16)]
        outv[pl.ds(g * 16, 16)] = acc

    pltpu.sync_copy(outv, out_hbm.at[row])


def _entropy_body(h_ref, o_ref):
    h = h_ref[...]
    total = jnp.sum(h, axis=1, keepdims=True)
    p = h / total
    logp = jnp.log2(jnp.where(p == 0, jnp.float32(1.0), p))
    ent = jnp.sum(-p * logp, axis=1)
    o_ref[...] = jnp.full((8, 128), jnp.mean(ent), jnp.float32)


def _entropy(hists):
    return pl.pallas_call(
        _entropy_body,
        in_specs=[pl.BlockSpec((B, BINS), lambda: (0, 0))],
        out_specs=pl.BlockSpec((8, 128), lambda: (0, 0)),
        out_shape=jax.ShapeDtypeStruct((8, 128), jnp.float32),
    )(hists)


def kernel(y_pred):
    y3 = y_pred.reshape(B, N // 128, 128)
    mn_b, scale_b = _minmax(y3)
    hists = _hist_sc(y_pred, mn_b.reshape(B, 128), scale_b.reshape(B, 128))
    out = _entropy(hists)
    return out[0, 0]


# speedup vs baseline: 137.7734x; 3.3331x over previous
import functools

import jax
import jax.numpy as jnp
from jax import lax
from jax.experimental import pallas as pl
from jax.experimental.pallas import tpu as pltpu
from jax.experimental.pallas import tpu_sc as plsc

B = 32
N = 1048576
BINS = 256
NCOPY = 16
CH = 32768
NCHUNK = N // CH
VREGS = CH // 16


def _minmax_body(x_ref, mn_ref, scale_ref):
    x = x_ref[...]
    mn = jnp.min(x)
    mx = jnp.max(x)
    rng = mx - mn
    safe = jnp.where(rng > 0, rng, jnp.float32(1.0))
    scale = jnp.float32(256.0) / safe
    mn_ref[...] = jnp.full((1, 1, 128), mn, jnp.float32)
    scale_ref[...] = jnp.full((1, 1, 128), scale, jnp.float32)


def _minmax(y3):
    return pl.pallas_call(
        _minmax_body,
        grid=(B,),
        in_specs=[pl.BlockSpec((1, N // 128, 128), lambda i: (i, 0, 0))],
        out_specs=[
            pl.BlockSpec((1, 1, 128), lambda i: (i, 0, 0)),
            pl.BlockSpec((1, 1, 128), lambda i: (i, 0, 0)),
        ],
        out_shape=[
            jax.ShapeDtypeStruct((B, 1, 128), jnp.float32),
            jax.ShapeDtypeStruct((B, 1, 128), jnp.float32),
        ],
    )(y3)


_mesh = plsc.VectorSubcoreMesh(core_axis_name="c", subcore_axis_name="s")


@functools.partial(
    pl.kernel,
    mesh=_mesh,
    compiler_params=pltpu.CompilerParams(needs_layout_passes=False),
    out_type=jax.ShapeDtypeStruct((B, BINS), jnp.float32),
    scratch_types=[
        pltpu.VMEM((CH,), jnp.float32),
        pltpu.VMEM((CH,), jnp.float32),
        pltpu.VMEM((BINS * NCOPY,), jnp.float32),
        pltpu.VMEM((BINS,), jnp.float32),
        pltpu.VMEM((16,), jnp.float32),
        pltpu.VMEM((16,), jnp.float32),
        pltpu.SemaphoreType.DMA,
        pltpu.SemaphoreType.DMA,
    ],
)
def _hist_sc(y_hbm, mn_hbm, scale_hbm, out_hbm,
             buf0, buf1, hist, outv, mnv, scv, sem0, sem1):
    row = lax.axis_index("s") * 2 + lax.axis_index("c")

    pltpu.sync_copy(mn_hbm.at[row, pl.ds(0, 16)], mnv)
    pltpu.sync_copy(scale_hbm.at[row, pl.ds(0, 16)], scv)

    zeros16 = jnp.zeros((16,), jnp.float32)

    @plsc.parallel_loop(0, (BINS * NCOPY) // 16, unroll=8)
    def _(i):
        hist[pl.ds(i * 16, 16)] = zeros16

    mn_v = mnv[...]
    sc_v = scv[...]
    lane_base = lax.iota(jnp.int32, 16) * 256
    ones = jnp.ones((16,), jnp.float32)
    cap = jnp.full((16,), 255, jnp.int32)

    bufs = (buf0, buf1)
    sems = (sem0, sem1)

    def chunk_body(buf, _):
        @plsc.parallel_loop(0, VREGS, unroll=8)
        def _(i):
            x = buf[pl.ds(i * 16, 16)]
            t = (x - mn_v) * sc_v
            idx = jnp.minimum(t.astype(jnp.int32), cap)
            plsc.addupdate_scatter(hist, [idx + lane_base], ones)

    handles = [pltpu.async_copy(y_hbm.at[row, pl.ds(0, CH)], buf0, sem0)]
    for c in range(NCHUNK):
        handles[c].wait()
        if c + 1 < NCHUNK:
            handles.append(
                pltpu.async_copy(
                    y_hbm.at[row, pl.ds((c + 1) * CH, CH)],
                    bufs[(c + 1) % 2],
                    sems[(c + 1) % 2],
                )
            )
        chunk_body(bufs[c % 2], None)

    for g in range(BINS // 16):
        acc = hist[pl.ds(g * 16, 16)]
        for cpy in range(1, NCOPY):
            acc = acc + hist[pl.ds(cpy * BINS + g * 16, 16)]
        outv[pl.ds(g * 16, 16)] = acc

    pltpu.sync_copy(outv, out_hbm.at[row])


def _entropy_body(h_ref, o_ref):
    h = h_ref[...]
    total = jnp.sum(h, axis=1, keepdims=True)
    p = h / total
    logp = jnp.log2(jnp.where(p == 0, jnp.float32(1.0), p))
    ent = jnp.sum(-p * logp, axis=1)
    o_ref[...] = jnp.full((8, 128), jnp.mean(ent), jnp.float32)


def _entropy(hists):
    return pl.pallas_call(
        _entropy_body,
        in_specs=[pl.BlockSpec((B, BINS), lambda: (0, 0))],
        out_specs=pl.BlockSpec((8, 128), lambda: (0, 0)),
        out_shape=jax.ShapeDtypeStruct((8, 128), jnp.float32),
    )(hists)


def kernel(y_pred):
    y3 = y_pred.reshape(B, N // 128, 128)
    mn_b, scale_b = _minmax(y3)
    hists = _hist_sc(y_pred, mn_b.reshape(B, 128), scale_b.reshape(B, 128))
    out = _entropy(hists)
    return out[0, 0]


# speedup vs baseline: 137.8927x; 1.0009x over previous
import functools

import jax
import jax.numpy as jnp
from jax import lax
from jax.experimental import pallas as pl
from jax.experimental.pallas import tpu as pltpu
from jax.experimental.pallas import tpu_sc as plsc

B = 32
N = 1048576
BINS = 256
NCOPY = 16
CH = 32768
NCHUNK = N // CH
VREGS = CH // 16


def _minmax_body(x_ref, mn_ref, scale_ref):
    x = x_ref[...]
    mn = jnp.min(x)
    mx = jnp.max(x)
    rng = mx - mn
    safe = jnp.where(rng > 0, rng, jnp.float32(1.0))
    scale = jnp.float32(256.0) / safe
    mn_ref[...] = jnp.full((1, 1, 128), mn, jnp.float32)
    scale_ref[...] = jnp.full((1, 1, 128), scale, jnp.float32)


def _minmax(y3):
    return pl.pallas_call(
        _minmax_body,
        grid=(B,),
        in_specs=[pl.BlockSpec((1, N // 128, 128), lambda i: (i, 0, 0))],
        out_specs=[
            pl.BlockSpec((1, 1, 128), lambda i: (i, 0, 0)),
            pl.BlockSpec((1, 1, 128), lambda i: (i, 0, 0)),
        ],
        out_shape=[
            jax.ShapeDtypeStruct((B, 1, 128), jnp.float32),
            jax.ShapeDtypeStruct((B, 1, 128), jnp.float32),
        ],
    )(y3)


_mesh = plsc.VectorSubcoreMesh(core_axis_name="c", subcore_axis_name="s")


@functools.partial(
    pl.kernel,
    mesh=_mesh,
    compiler_params=pltpu.CompilerParams(
        needs_layout_passes=False, use_tc_tiling_on_sc=True
    ),
    out_type=jax.ShapeDtypeStruct((B, BINS), jnp.float32),
    scratch_types=[
        pltpu.VMEM((CH,), jnp.float32),
        pltpu.VMEM((CH,), jnp.float32),
        pltpu.VMEM((BINS * NCOPY,), jnp.float32),
        pltpu.VMEM((BINS,), jnp.float32),
        pltpu.VMEM((16,), jnp.float32),
        pltpu.VMEM((16,), jnp.float32),
        pltpu.SemaphoreType.DMA,
        pltpu.SemaphoreType.DMA,
    ],
)
def _hist_sc(y_hbm, mn_hbm, scale_hbm, out_hbm,
             buf0, buf1, hist, outv, mnv, scv, sem0, sem1):
    row = lax.axis_index("s") * 2 + lax.axis_index("c")

    pltpu.sync_copy(mn_hbm.at[row, pl.ds(0, 16)], mnv)
    pltpu.sync_copy(scale_hbm.at[row, pl.ds(0, 16)], scv)

    zeros16 = jnp.zeros((16,), jnp.float32)

    @plsc.parallel_loop(0, (BINS * NCOPY) // 16, unroll=8)
    def _(i):
        hist[pl.ds(i * 16, 16)] = zeros16

    mn_v = mnv[...]
    sc_v = scv[...]
    lane_base = lax.iota(jnp.int32, 16) * 256
    ones = jnp.ones((16,), jnp.float32)
    cap = jnp.full((16,), 255, jnp.int32)

    bufs = (buf0, buf1)
    sems = (sem0, sem1)

    def chunk_body(buf, _):
        @plsc.parallel_loop(0, VREGS, unroll=8)
        def _(i):
            x = buf[pl.ds(i * 16, 16)]
            t = (x - mn_v) * sc_v
            idx = jnp.minimum(t.astype(jnp.int32), cap)
            plsc.addupdate_scatter(hist, [idx + lane_base], ones)

    handles = [pltpu.async_copy(y_hbm.at[row, pl.ds(0, CH)], buf0, sem0)]
    for c in range(NCHUNK):
        handles[c].wait()
        if c + 1 < NCHUNK:
            handles.append(
                pltpu.async_copy(
                    y_hbm.at[row, pl.ds((c + 1) * CH, CH)],
                    bufs[(c + 1) % 2],
                    sems[(c + 1) % 2],
                )
            )
        chunk_body(bufs[c % 2], None)

    for g in range(BINS // 16):
        acc = hist[pl.ds(g * 16, 16)]
        for cpy in range(1, NCOPY):
            acc = acc + hist[pl.ds(cpy * BINS + g * 16, 16)]
        outv[pl.ds(g * 16, 16)] = acc

    pltpu.sync_copy(outv, out_hbm.at[row])


def _entropy_body(h_ref, o_ref):
    h = h_ref[...]
    total = jnp.sum(h, axis=1, keepdims=True)
    p = h / total
    logp = jnp.log2(jnp.where(p == 0, jnp.float32(1.0), p))
    ent = jnp.sum(-p * logp, axis=1)
    o_ref[...] = jnp.full((8, 128), jnp.mean(ent), jnp.float32)


def _entropy(hists):
    return pl.pallas_call(
        _entropy_body,
        in_specs=[pl.BlockSpec((B, BINS), lambda: (0, 0))],
        out_specs=pl.BlockSpec((8, 128), lambda: (0, 0)),
        out_shape=jax.ShapeDtypeStruct((8, 128), jnp.float32),
    )(hists)


def kernel(y_pred):
    y3 = y_pred.reshape(B, N // 128, 128)
    mn_b, scale_b = _minmax(y3)
    hists = _hist_sc(y_pred, mn_b.reshape(B, 128), scale_b.reshape(B, 128))
    out = _entropy(hists)
    return out[0, 0]
